# Initial kernel scaffold; baseline (speedup 1.0000x reference)
import jax
import jax.numpy as jnp
from jax.experimental import pallas as pl


def kernel(inputs, noise, Wp, bp):
    raise NotImplementedError("write your pallas kernel here")


# speedup vs baseline: 1.2715x; 1.2715x over previous
import functools

import jax
import jax.numpy as jnp
from jax import lax
from jax.experimental import pallas as pl
from jax.experimental.pallas import tpu as pltpu
from jax.experimental.pallas import tpu_sc as plsc

N = 32
C = 3
H = W = 512
BLK = 16
G = H // BLK
CHUNK = BLK * W
NC, NS = 2, 16


def _sc_body(x_hbm, w_hbm, out_hbm, wv, buf, out_v, sem_a, sem_b):
    cid = lax.axis_index("c")
    sid = lax.axis_index("s")
    n = sid * NC + cid

    pltpu.sync_copy(w_hbm, wv)
    wvec = wv[...]
    wsplat = [jnp.full((16,), wvec[c]) for c in range(C)]

    def chunk_src(c, g0):
        off = ((n * C + c) * G + g0) * CHUNK
        return x_hbm.at[pl.ds(off, CHUNK)]

    def issue(g0, base, sem):
        for c in range(C):
            pltpu.make_async_copy(chunk_src(c, g0), buf.at[base + c], sem).start()

    def process(g0, base, sem):
        for c in range(C):
            pltpu.make_async_copy(chunk_src(c, g0), buf.at[base + c], sem).wait()

        def rbody(r, acc):
            acc = list(acc)
            for c in range(C):
                for g1 in range(G):
                    v = buf[base + c, pl.ds(r * W + g1 * BLK, BLK)]
                    acc[g1] = acc[g1] + wsplat[c] * v
            return tuple(acc)

        acc = lax.fori_loop(
            0, BLK, rbody,
            tuple(jnp.zeros((16,), jnp.float32) for _ in range(G)),
        )

        lane = lax.broadcasted_iota(jnp.int32, (16,), 0)
        for h in range(2):
            vec = jnp.zeros((16,), jnp.float32)
            for j in range(16):
                s = jnp.sum(acc[h * 16 + j])
                vec = jnp.where(lane == j, s, vec)
            out_v[pl.ds(g0 * G + h * 16, 16)] = vec

    issue(0, 0, sem_a)
    issue(1, C, sem_b)

    def pbody(p, carry):
        g0a = 2 * p
        process(g0a, 0, sem_a)

        @pl.when(p < G // 2 - 1)
        def _():
            issue(g0a + 2, 0, sem_a)

        process(g0a + 1, C, sem_b)

        @pl.when(p < G // 2 - 1)
        def _():
            issue(g0a + 3, C, sem_b)

        return carry

    lax.fori_loop(0, G // 2, pbody, 0)

    pltpu.sync_copy(out_v, out_hbm.at[n])


@jax.jit
def _sc_pool(x_flat, wpad):
    mesh = plsc.VectorSubcoreMesh(core_axis_name="c", subcore_axis_name="s")
    return pl.kernel(
        _sc_body,
        out_type=jax.ShapeDtypeStruct((N, G * G), jnp.float32),
        mesh=mesh,
        scratch_types=[
            pltpu.VMEM((16,), jnp.float32),
            pltpu.VMEM((2 * C, CHUNK), jnp.float32),
            pltpu.VMEM((G * G,), jnp.float32),
            pltpu.SemaphoreType.DMA,
            pltpu.SemaphoreType.DMA,
        ],
        compiler_params=pltpu.CompilerParams(needs_layout_passes=False),
    )(x_flat, wpad)


def _tc_body(s_ref, nz_ref, bp_ref, grid_ref, probs_ref, lp_ref, cnt_ref):
    logits = s_ref[...] + bp_ref[0]
    probs = jax.nn.sigmoid(logits)
    grid_f = jnp.where(nz_ref[...] < probs, 1.0, 0.0).astype(jnp.float32)
    lp = grid_f * jax.nn.log_sigmoid(logits) + (1.0 - grid_f) * jax.nn.log_sigmoid(-logits)
    grid_ref[...] = grid_f
    probs_ref[...] = probs
    lp_ref[...] = lp
    cnt_ref[0] = jnp.sum(grid_f) * (1.0 / (N * G * G))


@jax.jit
def _tc_finish(s, nz, bp):
    m = N * G * G // 128
    return pl.pallas_call(
        _tc_body,
        out_shape=(
            jax.ShapeDtypeStruct((m, 128), jnp.float32),
            jax.ShapeDtypeStruct((m, 128), jnp.float32),
            jax.ShapeDtypeStruct((m, 128), jnp.float32),
            jax.ShapeDtypeStruct((1,), jnp.float32),
        ),
        in_specs=[
            pl.BlockSpec(memory_space=pltpu.VMEM),
            pl.BlockSpec(memory_space=pltpu.VMEM),
            pl.BlockSpec(memory_space=pltpu.SMEM),
        ],
        out_specs=(
            pl.BlockSpec(memory_space=pltpu.VMEM),
            pl.BlockSpec(memory_space=pltpu.VMEM),
            pl.BlockSpec(memory_space=pltpu.VMEM),
            pl.BlockSpec(memory_space=pltpu.SMEM),
        ),
    )(s, nz, bp)


def kernel(inputs, noise, Wp, bp):
    x_flat = inputs.reshape(-1)
    wpad = jnp.zeros((16,), jnp.float32).at[:C].set(Wp[0] / (BLK * BLK))
    s = _sc_pool(x_flat, wpad)

    m = N * G * G // 128
    grid_f, probs, lp, perc = _tc_finish(
        s.reshape(m, 128), noise.reshape(m, 128), bp.reshape(1)
    )
    shp = (N, 1, G, G)
    return (
        grid_f.reshape(shp).astype(bool),
        probs.reshape(shp),
        lp.reshape(shp),
        perc.reshape(()),
    )


# speedup vs baseline: 2.6923x; 2.1173x over previous
import functools

import jax
import jax.numpy as jnp
from jax import lax
from jax.experimental import pallas as pl
from jax.experimental.pallas import tpu as pltpu
from jax.experimental.pallas import tpu_sc as plsc

N = 32
C = 3
H = W = 512
BLK = 16
G = H // BLK
NC, NS = 2, 16
GS = 16
GR = G - GS


def _sc_body(x_hbm, out_hbm, buf, out_v, sem_a, sem_b):
    cid = lax.axis_index("c")
    sid = lax.axis_index("s")
    n = sid * NC + cid

    def chunk_src(c, g0):
        return x_hbm.at[n, c, pl.ds(g0 * BLK, BLK), :]

    def issue(g0, base, sem):
        for c in range(C):
            pltpu.make_async_copy(chunk_src(c, g0), buf.at[base + c], sem).start()

    def process(g0, base, sem):
        for c in range(C):
            pltpu.make_async_copy(chunk_src(c, g0), buf.at[base + c], sem).wait()

        lane = lax.broadcasted_iota(jnp.int32, (16,), 0)
        for c in range(C):
            def rbody(r, acc, c=c):
                acc = list(acc)
                for g1 in range(G):
                    acc[g1] = acc[g1] + buf[base + c, r, pl.ds(g1 * BLK, BLK)]
                return tuple(acc)

            acc = lax.fori_loop(
                0, BLK, rbody,
                tuple(jnp.zeros((16,), jnp.float32) for _ in range(G)),
            )

            for h in range(2):
                vec = jnp.zeros((16,), jnp.float32)
                for j in range(16):
                    s = jnp.sum(acc[h * 16 + j])
                    vec = jnp.where(lane == j, s, vec)
                out_v[pl.ds(c * (GR * G) + (g0 - GS) * G + h * 16, 16)] = vec

    issue(GS, 0, sem_a)
    issue(GS + 1, C, sem_b)
    npair = GR // 2

    def pbody(p, carry):
        g0a = GS + 2 * p
        process(g0a, 0, sem_a)

        @pl.when(p < npair - 1)
        def _():
            issue(g0a + 2, 0, sem_a)

        process(g0a + 1, C, sem_b)

        @pl.when(p < npair - 1)
        def _():
            issue(g0a + 3, C, sem_b)

        return carry

    lax.fori_loop(0, npair, pbody, 0)

    pltpu.sync_copy(out_v, out_hbm.at[n])


@jax.jit
def _sc_pool(x):
    mesh = plsc.VectorSubcoreMesh(core_axis_name="c", subcore_axis_name="s")
    return pl.kernel(
        _sc_body,
        out_type=jax.ShapeDtypeStruct((N, C * GR * G), jnp.float32),
        mesh=mesh,
        scratch_types=[
            pltpu.VMEM((2 * C, BLK, W), jnp.float32),
            pltpu.VMEM((C * GR * G,), jnp.float32),
            pltpu.SemaphoreType.DMA,
            pltpu.SemaphoreType.DMA,
        ],
        compiler_params=pltpu.CompilerParams(needs_layout_passes=False),
    )(x)


def _tc_pool_body(x_ref, a_ref, b_ref, out_ref):
    x = x_ref[0].reshape(C * GS * BLK, W)
    y = jax.lax.dot(a_ref[...], x, precision=jax.lax.Precision.HIGHEST,
                    preferred_element_type=jnp.float32)
    z = jax.lax.dot(y, b_ref[...], precision=jax.lax.Precision.HIGHEST,
                    preferred_element_type=jnp.float32)
    out_ref[0] = z.reshape(C, GS, G)


@jax.jit
def _tc_pool(x, a, b):
    return pl.pallas_call(
        _tc_pool_body,
        grid=(N,),
        in_specs=[
            pl.BlockSpec((1, C, GS * BLK, W), lambda n: (n, 0, 0, 0)),
            pl.BlockSpec((C * GS, C * GS * BLK), lambda n: (0, 0)),
            pl.BlockSpec((W, G), lambda n: (0, 0)),
        ],
        out_specs=pl.BlockSpec((1, C, GS, G), lambda n: (n, 0, 0, 0)),
        out_shape=jax.ShapeDtypeStruct((N, C, GS, G), jnp.float32),
    )(x, a, b)


def _round_bf16(x):
    b = lax.bitcast_convert_type(x, jnp.uint32)
    b = (b + jnp.uint32(0x7FFF) + ((b >> 16) & jnp.uint32(1))) & jnp.uint32(0xFFFF0000)
    return lax.bitcast_convert_type(b, jnp.float32)


def _tc_body(s_ref, nz_ref, wb_ref, bp_ref, grid_ref, probs_ref, lp_ref, cnt_ref):
    m = [_round_bf16(s_ref[c] * (1.0 / (BLK * BLK))) for c in range(C)]
    logits = (m[0] * wb_ref[0] + m[1] * wb_ref[1]) + m[2] * wb_ref[2] + bp_ref[0]
    probs = jax.nn.sigmoid(logits)
    grid_f = jnp.where(nz_ref[...] < probs, 1.0, 0.0).astype(jnp.float32)
    lp = grid_f * jax.nn.log_sigmoid(logits) + (1.0 - grid_f) * jax.nn.log_sigmoid(-logits)
    grid_ref[...] = grid_f
    probs_ref[...] = probs
    lp_ref[...] = lp
    cnt_ref[0] = jnp.sum(grid_f) * (1.0 / (N * G * G))


@jax.jit
def _tc_finish(s3, nz, wb, bp):
    m = N * G * G // 128
    return pl.pallas_call(
        _tc_body,
        out_shape=(
            jax.ShapeDtypeStruct((m, 128), jnp.float32),
            jax.ShapeDtypeStruct((m, 128), jnp.float32),
            jax.ShapeDtypeStruct((m, 128), jnp.float32),
            jax.ShapeDtypeStruct((1,), jnp.float32),
        ),
        in_specs=[
            pl.BlockSpec(memory_space=pltpu.VMEM),
            pl.BlockSpec(memory_space=pltpu.VMEM),
            pl.BlockSpec(memory_space=pltpu.SMEM),
            pl.BlockSpec(memory_space=pltpu.SMEM),
        ],
        out_specs=(
            pl.BlockSpec(memory_space=pltpu.VMEM),
            pl.BlockSpec(memory_space=pltpu.VMEM),
            pl.BlockSpec(memory_space=pltpu.VMEM),
            pl.BlockSpec(memory_space=pltpu.SMEM),
        ),
    )(s3, nz, wb, bp)


def _pool_mats():
    rows = jnp.arange(GS * BLK) // BLK
    asel = (jnp.arange(GS)[:, None] == rows[None, :]).astype(jnp.float32)
    a = jnp.kron(jnp.eye(C, dtype=jnp.float32), asel)
    b = (jnp.arange(W)[:, None] // BLK == jnp.arange(G)[None, :]).astype(jnp.float32)
    return a, b


def kernel(inputs, noise, Wp, bp):
    a, b = _pool_mats()
    s_tc = _tc_pool(inputs, a, b)
    s_sc = _sc_pool(inputs).reshape(N, C, GR, G)
    s = jnp.concatenate([s_tc, s_sc], axis=2)
    m = N * G * G // 128
    s3 = s.transpose(1, 0, 2, 3).reshape(C, m, 128)
    wb = lax.reduce_precision(Wp[0], 8, 7)

    grid_f, probs, lp, perc = _tc_finish(
        s3, noise.reshape(m, 128), wb, bp.reshape(1)
    )
    shp = (N, 1, G, G)
    return (
        grid_f.reshape(shp).astype(bool),
        probs.reshape(shp),
        lp.reshape(shp),
        perc.reshape(()),
    )
